# Initial kernel scaffold; baseline (speedup 1.0000x reference)
import jax
import jax.numpy as jnp
from jax.experimental import pallas as pl


def kernel(A_star, edge_index, node_mask):
    raise NotImplementedError("write your pallas kernel here")


# speedup vs baseline: 12.1729x; 12.1729x over previous
import functools

import jax
import jax.numpy as jnp
from jax import lax
from jax.experimental import pallas as pl
from jax.experimental.pallas import tpu as pltpu
from jax.experimental.pallas import tpu_sc as plsc

N_NODES = 10000
D_FEAT = 128
N_EDGES = 160000
EPS = 1e-15
TABLE_SIZE = N_NODES * N_NODES

NUM_CORES = 2
NUM_SUBCORES = 16
NW = NUM_CORES * NUM_SUBCORES
ROWS_PER_TILE = 40
E_PER_TILE = ROWS_PER_TILE * 128
E_PAD = NW * E_PER_TILE
ROWS_TOTAL = E_PAD // 128
BIG_SCORE = 40.0
K_FIRE = 8


def _mesh():
    return plsc.VectorSubcoreMesh(
        core_axis_name="c", subcore_axis_name="s",
        num_cores=NUM_CORES, num_subcores=NUM_SUBCORES)


def _wid():
    return lax.axis_index("s") * NUM_CORES + lax.axis_index("c")


def _compute_keys(src_v, dst_v, keys_v, vals_v, base_eid):
    def row(g, _):
        def chunk(cc, _):
            off = pl.multiple_of(cc * 16, 16)
            s = src_v[g, pl.ds(off, 16)]
            d = dst_v[g, pl.ds(off, 16)]
            keys_v[g, pl.ds(off, 16)] = s * N_NODES + d
            vals_v[g, pl.ds(off, 16)] = (
                base_eid + g * 128 + cc * 16 + lax.iota(jnp.int32, 16))
            return _
        return lax.fori_loop(0, 8, chunk, None)
    lax.fori_loop(0, ROWS_PER_TILE, row, None)


@functools.partial(
    pl.kernel,
    out_type=jax.ShapeDtypeStruct((TABLE_SIZE,), jnp.int32),
    mesh=_mesh(),
    compiler_params=pltpu.CompilerParams(needs_layout_passes=False),
    scratch_types=[
        pltpu.VMEM((ROWS_PER_TILE, 128), jnp.int32),
        pltpu.VMEM((ROWS_PER_TILE, 128), jnp.int32),
        pltpu.VMEM((ROWS_PER_TILE, 128), jnp.int32),
        pltpu.VMEM((ROWS_PER_TILE, 128), jnp.int32),
        pltpu.SemaphoreType.DMA,
    ],
)
def _scatter_ids(src_hbm, dst_hbm, table_hbm, src_v, dst_v, keys_v, vals_v,
                 sem):
    wid = _wid()
    row0 = wid * ROWS_PER_TILE
    pltpu.sync_copy(src_hbm.at[pl.ds(row0, ROWS_PER_TILE)], src_v)
    pltpu.sync_copy(dst_hbm.at[pl.ds(row0, ROWS_PER_TILE)], dst_v)
    _compute_keys(src_v, dst_v, keys_v, vals_v, row0 * 128)

    def scat(gg, _):
        cps = [
            pltpu.async_copy(vals_v.at[gg * K_FIRE + j],
                             table_hbm.at[keys_v.at[gg * K_FIRE + j]], sem)
            for j in range(K_FIRE)
        ]
        for c in cps:
            c.wait()
        return _
    lax.fori_loop(0, ROWS_PER_TILE // K_FIRE, scat, None)


@functools.partial(
    pl.kernel,
    out_type=jax.ShapeDtypeStruct((ROWS_TOTAL, 128), jnp.float32),
    mesh=_mesh(),
    compiler_params=pltpu.CompilerParams(needs_layout_passes=False),
    scratch_types=[
        pltpu.VMEM((ROWS_PER_TILE, 128), jnp.int32),
        pltpu.VMEM((ROWS_PER_TILE, 128), jnp.int32),
        pltpu.VMEM((ROWS_PER_TILE, 128), jnp.int32),
        pltpu.VMEM((ROWS_PER_TILE, 128), jnp.int32),
        pltpu.VMEM((ROWS_PER_TILE, 128), jnp.float32),
        pltpu.VMEM((N_NODES,), jnp.int32),
        pltpu.VMEM((128, D_FEAT), jnp.float32),
        pltpu.VMEM((128, D_FEAT), jnp.float32),
        pltpu.SemaphoreType.DMA,
        pltpu.SemaphoreType.DMA,
    ],
)
def _gather_dot(src_hbm, dst_hbm, mask_hbm, a_hbm, table_hbm, out_hbm,
                src_v, dst_v, keys_v, rep_v, scores_v, mask_v,
                rows_s, rows_d, sem_a, sem_b):
    wid = _wid()
    row0 = wid * ROWS_PER_TILE
    base_eid = row0 * 128
    pltpu.sync_copy(src_hbm.at[pl.ds(row0, ROWS_PER_TILE)], src_v)
    pltpu.sync_copy(dst_hbm.at[pl.ds(row0, ROWS_PER_TILE)], dst_v)
    pltpu.sync_copy(mask_hbm, mask_v)
    _compute_keys(src_v, dst_v, keys_v, rep_v, base_eid)

    def rget(gg, _):
        cps = [
            pltpu.async_copy(table_hbm.at[keys_v.at[gg * K_FIRE + j]],
                             rep_v.at[gg * K_FIRE + j], sem_a)
            for j in range(K_FIRE)
        ]
        for c in cps:
            c.wait()
        return _
    lax.fori_loop(0, ROWS_PER_TILE // K_FIRE, rget, None)

    def grp(g, _):
        cs = pltpu.async_copy(a_hbm.at[src_v.at[g]], rows_s, sem_a)
        cd = pltpu.async_copy(a_hbm.at[dst_v.at[g]], rows_d, sem_b)
        cs.wait()
        cd.wait()

        def sub(bb, _):
            b0 = pl.multiple_of(bb * 16, 16)
            eidx = b0 + lax.iota(jnp.int32, 16)
            sv = jnp.zeros((16,), jnp.float32)
            for d in range(D_FEAT):
                dsplat = jnp.full((16,), d, jnp.int32)
                gs = plsc.load_gather(rows_s, [eidx, dsplat])
                gd = plsc.load_gather(rows_d, [eidx, dsplat])
                sv = sv + gs * gd
            eid = (base_eid + g * 128 + bb * 16 + lax.iota(jnp.int32, 16))
            ms = plsc.load_gather(mask_v, [src_v[g, pl.ds(b0, 16)]])
            md = plsc.load_gather(mask_v, [dst_v[g, pl.ds(b0, 16)]])
            rep = rep_v[g, pl.ds(b0, 16)]
            valid = (rep == eid) & (ms != md)
            scores_v[g, pl.ds(b0, 16)] = jnp.where(valid, sv, BIG_SCORE)
            return _
        lax.fori_loop(0, 8, sub, None)
        return _
    lax.fori_loop(0, ROWS_PER_TILE, grp, None)
    pltpu.sync_copy(scores_v, out_hbm.at[pl.ds(row0, ROWS_PER_TILE)])


def _tc_loss_body(scores_ref, out_ref):
    s = scores_ref[...]
    terms = -jnp.log(jax.nn.sigmoid(s) + EPS)
    out_ref[0, 0] = jnp.sum(terms)


_tc_loss = pl.pallas_call(
    _tc_loss_body,
    out_shape=jax.ShapeDtypeStruct((1, 1), jnp.float32),
    out_specs=pl.BlockSpec(memory_space=pltpu.SMEM),
)


def kernel(A_star, edge_index, node_mask):
    ei = edge_index.astype(jnp.int32)
    src = jnp.pad(ei[0], (0, E_PAD - N_EDGES)).reshape(ROWS_TOTAL, 128)
    dst = jnp.pad(ei[1], (0, E_PAD - N_EDGES)).reshape(ROWS_TOTAL, 128)
    mask_i = node_mask.astype(jnp.int32)
    table = _scatter_ids(src, dst)
    scores = _gather_dot(src, dst, mask_i, A_star, table)
    return _tc_loss(scores)[0, 0]


# speedup vs baseline: 16.1683x; 1.3282x over previous
import functools

import jax
import jax.numpy as jnp
from jax import lax
from jax.experimental import pallas as pl
from jax.experimental.pallas import tpu as pltpu
from jax.experimental.pallas import tpu_sc as plsc

N_NODES = 10000
D_FEAT = 128
N_EDGES = 160000
EPS = 1e-15
TABLE_SIZE = N_NODES * N_NODES

NUM_CORES = 2
NUM_SUBCORES = 16
NW = NUM_CORES * NUM_SUBCORES
ROWS_PER_TILE = 40
E_PER_TILE = ROWS_PER_TILE * 128
E_PAD = NW * E_PER_TILE
ROWS_TOTAL = E_PAD // 128
BIG_SCORE = 40.0
K_FIRE = 8


def _mesh():
    return plsc.VectorSubcoreMesh(
        core_axis_name="c", subcore_axis_name="s",
        num_cores=NUM_CORES, num_subcores=NUM_SUBCORES)


def _wid():
    return lax.axis_index("s") * NUM_CORES + lax.axis_index("c")


def _compute_keys(src_v, dst_v, keys_v, vals_v, base_eid):
    def row(g, _):
        def chunk(cc, _):
            off = pl.multiple_of(cc * 16, 16)
            s = src_v[g, pl.ds(off, 16)]
            d = dst_v[g, pl.ds(off, 16)]
            keys_v[g, pl.ds(off, 16)] = s * N_NODES + d
            vals_v[g, pl.ds(off, 16)] = (
                base_eid + g * 128 + cc * 16 + lax.iota(jnp.int32, 16))
            return _
        return lax.fori_loop(0, 8, chunk, None)
    lax.fori_loop(0, ROWS_PER_TILE, row, None)


@functools.partial(
    pl.kernel,
    out_type=jax.ShapeDtypeStruct((TABLE_SIZE,), jnp.int32),
    mesh=_mesh(),
    compiler_params=pltpu.CompilerParams(needs_layout_passes=False),
    scratch_types=[
        pltpu.VMEM((ROWS_PER_TILE, 128), jnp.int32),
        pltpu.VMEM((ROWS_PER_TILE, 128), jnp.int32),
        pltpu.VMEM((ROWS_PER_TILE, 128), jnp.int32),
        pltpu.VMEM((ROWS_PER_TILE, 128), jnp.int32),
        pltpu.SemaphoreType.DMA,
    ],
)
def _scatter_ids(src_hbm, dst_hbm, table_hbm, src_v, dst_v, keys_v, vals_v,
                 sem):
    wid = _wid()
    row0 = wid * ROWS_PER_TILE
    pltpu.sync_copy(src_hbm.at[pl.ds(row0, ROWS_PER_TILE)], src_v)
    pltpu.sync_copy(dst_hbm.at[pl.ds(row0, ROWS_PER_TILE)], dst_v)
    _compute_keys(src_v, dst_v, keys_v, vals_v, row0 * 128)

    cps = [pltpu.async_copy(vals_v.at[g], table_hbm.at[keys_v.at[g]], sem)
           for g in range(ROWS_PER_TILE)]
    for c in cps:
        c.wait()


@functools.partial(
    pl.kernel,
    out_type=jax.ShapeDtypeStruct((ROWS_TOTAL, 128), jnp.float32),
    mesh=_mesh(),
    compiler_params=pltpu.CompilerParams(needs_layout_passes=False),
    scratch_types=[
        pltpu.VMEM((ROWS_PER_TILE, 128), jnp.int32),
        pltpu.VMEM((ROWS_PER_TILE, 128), jnp.int32),
        pltpu.VMEM((ROWS_PER_TILE, 128), jnp.int32),
        pltpu.VMEM((ROWS_PER_TILE, 128), jnp.int32),
        pltpu.VMEM((ROWS_PER_TILE, 128), jnp.float32),
        pltpu.VMEM((N_NODES,), jnp.int32),
        pltpu.VMEM((128, D_FEAT), jnp.float32),
        pltpu.VMEM((128, D_FEAT), jnp.float32),
        pltpu.VMEM((128, D_FEAT), jnp.float32),
        pltpu.VMEM((128, D_FEAT), jnp.float32),
        pltpu.SemaphoreType.DMA,
        pltpu.SemaphoreType.DMA,
        pltpu.SemaphoreType.DMA,
        pltpu.SemaphoreType.DMA,
        pltpu.SemaphoreType.DMA,
    ],
)
def _gather_dot(src_hbm, dst_hbm, mask_hbm, a_hbm, table_hbm, out_hbm,
                src_v, dst_v, keys_v, rep_v, scores_v, mask_v,
                rows_s0, rows_d0, rows_s1, rows_d1,
                sem_a, sem_a0, sem_b0, sem_a1, sem_b1):
    wid = _wid()
    row0 = wid * ROWS_PER_TILE
    base_eid = row0 * 128
    pltpu.sync_copy(src_hbm.at[pl.ds(row0, ROWS_PER_TILE)], src_v)
    pltpu.sync_copy(dst_hbm.at[pl.ds(row0, ROWS_PER_TILE)], dst_v)
    pltpu.sync_copy(mask_hbm, mask_v)
    _compute_keys(src_v, dst_v, keys_v, rep_v, base_eid)

    cps = [pltpu.async_copy(table_hbm.at[keys_v.at[g]], rep_v.at[g], sem_a)
           for g in range(ROWS_PER_TILE)]
    for c in cps:
        c.wait()

    def start_grp(g, bs, bd, sa, sb):
        pltpu.async_copy(a_hbm.at[src_v.at[g]], bs, sa)
        pltpu.async_copy(a_hbm.at[dst_v.at[g]], bd, sb)

    def wait_grp(g, bs, bd, sa, sb):
        pltpu.make_async_copy(a_hbm.at[src_v.at[g]], bs, sa).wait()
        pltpu.make_async_copy(a_hbm.at[dst_v.at[g]], bd, sb).wait()

    def compute_grp(g, rows_s, rows_d):
        def sub(bb, _):
            b0 = pl.multiple_of(bb * 16, 16)
            eidx = b0 + lax.iota(jnp.int32, 16)
            sv = jnp.zeros((16,), jnp.float32)
            for d in range(D_FEAT):
                dsplat = jnp.full((16,), d, jnp.int32)
                gs = plsc.load_gather(rows_s, [eidx, dsplat])
                gd = plsc.load_gather(rows_d, [eidx, dsplat])
                sv = sv + gs * gd
            eid = (base_eid + g * 128 + bb * 16 + lax.iota(jnp.int32, 16))
            ms = plsc.load_gather(mask_v, [src_v[g, pl.ds(b0, 16)]])
            md = plsc.load_gather(mask_v, [dst_v[g, pl.ds(b0, 16)]])
            rep = rep_v[g, pl.ds(b0, 16)]
            valid = (rep == eid) & (ms != md)
            scores_v[g, pl.ds(b0, 16)] = jnp.where(valid, sv, BIG_SCORE)
            return _
        lax.fori_loop(0, 8, sub, None)

    start_grp(0, rows_s0, rows_d0, sem_a0, sem_b0)

    def grp2(gg, _):
        g0 = gg * 2
        g1 = g0 + 1
        start_grp(g1, rows_s1, rows_d1, sem_a1, sem_b1)
        wait_grp(g0, rows_s0, rows_d0, sem_a0, sem_b0)
        compute_grp(g0, rows_s0, rows_d0)

        @pl.when(g1 + 1 < ROWS_PER_TILE)
        def _prefetch():
            start_grp(g1 + 1, rows_s0, rows_d0, sem_a0, sem_b0)

        wait_grp(g1, rows_s1, rows_d1, sem_a1, sem_b1)
        compute_grp(g1, rows_s1, rows_d1)
        return _
    lax.fori_loop(0, ROWS_PER_TILE // 2, grp2, None)
    pltpu.sync_copy(scores_v, out_hbm.at[pl.ds(row0, ROWS_PER_TILE)])


def _tc_loss_body(scores_ref, out_ref):
    s = scores_ref[...]
    terms = -jnp.log(jax.nn.sigmoid(s) + EPS)
    out_ref[0, 0] = jnp.sum(terms)


_tc_loss = pl.pallas_call(
    _tc_loss_body,
    out_shape=jax.ShapeDtypeStruct((1, 1), jnp.float32),
    out_specs=pl.BlockSpec(memory_space=pltpu.SMEM),
)


def kernel(A_star, edge_index, node_mask):
    ei = edge_index.astype(jnp.int32)
    src = jnp.pad(ei[0], (0, E_PAD - N_EDGES)).reshape(ROWS_TOTAL, 128)
    dst = jnp.pad(ei[1], (0, E_PAD - N_EDGES)).reshape(ROWS_TOTAL, 128)
    mask_i = node_mask.astype(jnp.int32)
    table = _scatter_ids(src, dst)
    scores = _gather_dot(src, dst, mask_i, A_star, table)
    return _tc_loss(scores)[0, 0]
